# baseline (speedup 1.0000x reference)
import functools

import jax
import jax.numpy as jnp
from jax import lax
from jax.experimental import pallas as pl
from jax.experimental.pallas import tpu as pltpu
from jax.experimental.pallas import tpu_sc as plsc

N = 10000
NP = 10240
E = 320000
D = 128
NC = 2
NS = 16
NW = NC * NS
EPW = E // NW
CH = 125
ITERS = EPW // CH
NB = 4
KB = ITERS // NB
RPS = NP // NS
DEGW = 16

_mesh = plsc.VectorSubcoreMesh(core_axis_name="c", subcore_axis_name="s")


@functools.partial(
    pl.kernel,
    mesh=_mesh,
    out_type=jax.ShapeDtypeStruct((NC, NP, DEGW), jnp.float32),
    scratch_types=[
        pltpu.VMEM((NB, KB, CH), jnp.int32),
        pltpu.VMEM((128, DEGW), jnp.float32),
        pltpu.VMEM_SHARED((NP, DEGW), jnp.float32),
    ],
)
def _deg_kernel(ei_hbm, out_hbm, dst_all, ones_v, deg_sh):
    cid = lax.axis_index("c")
    sid = lax.axis_index("s")
    wid = cid * NS + sid
    pltpu.sync_copy(ei_hbm.at[1, wid], dst_all)
    one16 = jnp.full((DEGW,), 1.0, jnp.float32)
    for i in range(128):
        ones_v[i, :] = one16
    r0 = sid * RPS
    for k in range(RPS // 128):
        pltpu.sync_copy(ones_v, deg_sh.at[pl.ds(r0 + k * 128, 128)])
    plsc.subcore_barrier()

    for b in range(NB):
        def body(j, carry, b=b):
            pltpu.sync_copy(ones_v.at[pl.ds(0, CH)],
                            deg_sh.at[dst_all.at[b, j]], add=True)
            return carry

        lax.fori_loop(0, KB, body, 0)
    plsc.subcore_barrier()
    pltpu.sync_copy(deg_sh.at[pl.ds(r0, RPS)], out_hbm.at[cid, pl.ds(r0, RPS)])


@functools.partial(
    pl.kernel,
    mesh=_mesh,
    out_type=jax.ShapeDtypeStruct((NC, NP, D), jnp.float32),
    scratch_types=[
        pltpu.VMEM((KB, CH), jnp.int32),
        pltpu.VMEM((KB, CH), jnp.int32),
        pltpu.VMEM((CH, D), jnp.float32),
        pltpu.VMEM((CH, D), jnp.float32),
        pltpu.VMEM_SHARED((NP, D), jnp.float32),
        pltpu.SemaphoreType.DMA,
        pltpu.SemaphoreType.DMA,
    ],
)
def _scatter_kernel(h_hbm, ei_hbm, out_hbm, src_blk, dst_blk,
                    rows0, rows1, acc_sh, sem0, sem1):
    cid = lax.axis_index("c")
    sid = lax.axis_index("s")
    wid = cid * NS + sid
    r0 = sid * RPS
    pltpu.sync_copy(h_hbm.at[pl.ds(r0, RPS)], acc_sh.at[pl.ds(r0, RPS)])
    plsc.subcore_barrier()

    for b in range(NB):
        pltpu.sync_copy(ei_hbm.at[0, wid, b], src_blk)
        pltpu.sync_copy(ei_hbm.at[1, wid, b], dst_blk)
        pltpu.async_copy(h_hbm.at[src_blk.at[0]], rows0, sem0)
        pltpu.async_copy(h_hbm.at[src_blk.at[1]], rows1, sem1)

        def pair(i, carry):
            j0 = 2 * i
            pltpu.make_async_copy(h_hbm.at[src_blk.at[j0]], rows0,
                                  sem0).wait()
            pltpu.sync_copy(rows0, acc_sh.at[dst_blk.at[j0]], add=True)
            pltpu.async_copy(h_hbm.at[src_blk.at[j0 + 2]], rows0, sem0)
            j1 = j0 + 1
            pltpu.make_async_copy(h_hbm.at[src_blk.at[j1]], rows1,
                                  sem1).wait()
            pltpu.sync_copy(rows1, acc_sh.at[dst_blk.at[j1]], add=True)
            pltpu.async_copy(h_hbm.at[src_blk.at[j1 + 2]], rows1, sem1)
            return carry

        lax.fori_loop(0, KB // 2 - 1, pair, 0)
        jlast = KB - 2
        pltpu.make_async_copy(h_hbm.at[src_blk.at[jlast]], rows0, sem0).wait()
        pltpu.sync_copy(rows0, acc_sh.at[dst_blk.at[jlast]], add=True)
        pltpu.make_async_copy(h_hbm.at[src_blk.at[jlast + 1]], rows1,
                              sem1).wait()
        pltpu.sync_copy(rows1, acc_sh.at[dst_blk.at[jlast + 1]], add=True)
    plsc.subcore_barrier()
    pltpu.sync_copy(acc_sh.at[pl.ds(r0, RPS)], out_hbm.at[cid, pl.ds(r0, RPS)])


def _dinv_from_parts(p_ref):
    deg = p_ref[0, :, 0:1] + p_ref[1, :, 0:1] - 1.0
    return lax.rsqrt(deg)


def _matmul_pad_body(x_ref, w_ref, o_ref):
    h = jnp.dot(x_ref[...], w_ref[...], preferred_element_type=jnp.float32)
    o_ref[pl.ds(0, N), :] = h
    o_ref[pl.ds(N, NP - N), :] = jnp.zeros((NP - N, D), jnp.float32)


def _scale_body(h_ref, p_ref, o_ref):
    o_ref[...] = h_ref[...] * _dinv_from_parts(p_ref)


BR = 1280
GR = NP // BR

_row_spec = pl.BlockSpec((BR, D), lambda i: (i, 0))
_parts_spec = pl.BlockSpec((NC, BR, DEGW), lambda i: (0, i, 0))
_t_spec = pl.BlockSpec((NC, BR, D), lambda i: (0, i, 0))
_vec_spec = pl.BlockSpec((D,), lambda i: (0,))
_w_spec = pl.BlockSpec((D, D), lambda i: (0, 0))


def _layer2_body(t_ref, h_ref, p_ref, b_ref, w_ref, o_ref):
    dinv = _dinv_from_parts(p_ref)
    agg = t_ref[0] + t_ref[1] - h_ref[...]
    act = jnp.maximum(agg * dinv + b_ref[...], 0.0)
    o_ref[...] = jnp.dot(act, w_ref[...],
                         preferred_element_type=jnp.float32) * dinv


def _final_body(t_ref, h_ref, p_ref, b_ref, wfc_ref, bfc_ref, o_ref, s_ref):
    i = pl.program_id(0)
    dinv = _dinv_from_parts(p_ref)
    agg = t_ref[0] + t_ref[1] - h_ref[...]
    act = jnp.maximum(agg * dinv + b_ref[...], 0.0)
    row = lax.broadcasted_iota(jnp.int32, (BR, D), 0) + i * BR
    act = jnp.where(row < N, act, 0.0)
    part = jnp.sum(act, axis=0, keepdims=True)

    @pl.when(i == 0)
    def _():
        s_ref[...] = jnp.zeros((1, D), jnp.float32)

    s_ref[...] += part

    @pl.when(i == GR - 1)
    def _():
        m = s_ref[...] * (1.0 / N)
        o_ref[...] = jnp.dot(m, wfc_ref[...],
                             preferred_element_type=jnp.float32) + bfc_ref[...]


def kernel(x, edge_index, W1, b1, W2, b2, Wfc, bfc):
    ei = jnp.asarray(edge_index, jnp.int32).reshape(2, NW, NB, KB, CH)

    parts = _deg_kernel(ei)

    h1 = pl.pallas_call(
        _matmul_pad_body,
        out_shape=jax.ShapeDtypeStruct((NP, D), jnp.float32),
    )(x, W1)

    h1t = pl.pallas_call(
        _scale_body,
        grid=(GR,),
        in_specs=[_row_spec, _parts_spec],
        out_specs=_row_spec,
        out_shape=jax.ShapeDtypeStruct((NP, D), jnp.float32),
    )(h1, parts)

    t1 = _scatter_kernel(h1t, ei)

    h2t = pl.pallas_call(
        _layer2_body,
        grid=(GR,),
        in_specs=[_t_spec, _row_spec, _parts_spec, _vec_spec, _w_spec],
        out_specs=_row_spec,
        out_shape=jax.ShapeDtypeStruct((NP, D), jnp.float32),
    )(t1, h1t, parts, b1, W2)

    t2 = _scatter_kernel(h2t, ei)

    out = pl.pallas_call(
        _final_body,
        grid=(GR,),
        in_specs=[_t_spec, _row_spec, _parts_spec, _vec_spec, _w_spec,
                  _vec_spec],
        out_specs=pl.BlockSpec((1, D), lambda i: (0, 0)),
        out_shape=jax.ShapeDtypeStruct((1, D), jnp.float32),
        scratch_shapes=[pltpu.VMEM((1, D), jnp.float32)],
    )(t2, h2t, parts, b2, Wfc, bfc)
    return out


# speedup vs baseline: 1.0119x; 1.0119x over previous
import functools

import jax
import jax.numpy as jnp
from jax import lax
from jax.experimental import pallas as pl
from jax.experimental.pallas import tpu as pltpu
from jax.experimental.pallas import tpu_sc as plsc

N = 10000
NP = 10240
E = 320000
D = 128
NC = 2
NS = 16
NW = NC * NS
EPW = E // NW
CH = 100
ITERS = EPW // CH
NB = 5
KB = ITERS // NB
RPS = NP // NS
DEGW = 16

_mesh = plsc.VectorSubcoreMesh(core_axis_name="c", subcore_axis_name="s")


@functools.partial(
    pl.kernel,
    mesh=_mesh,
    out_type=jax.ShapeDtypeStruct((NC, NP, DEGW), jnp.float32),
    scratch_types=[
        pltpu.VMEM((NB, KB, CH), jnp.int32),
        pltpu.VMEM((128, DEGW), jnp.float32),
        pltpu.VMEM_SHARED((NP, DEGW), jnp.float32),
    ],
)
def _deg_kernel(ei_hbm, out_hbm, dst_all, ones_v, deg_sh):
    cid = lax.axis_index("c")
    sid = lax.axis_index("s")
    wid = cid * NS + sid
    pltpu.sync_copy(ei_hbm.at[1, wid], dst_all)
    one16 = jnp.full((DEGW,), 1.0, jnp.float32)
    for i in range(128):
        ones_v[i, :] = one16
    r0 = sid * RPS
    for k in range(RPS // 128):
        pltpu.sync_copy(ones_v, deg_sh.at[pl.ds(r0 + k * 128, 128)])
    plsc.subcore_barrier()

    for b in range(NB):
        def body(j, carry, b=b):
            pltpu.sync_copy(ones_v.at[pl.ds(0, CH)],
                            deg_sh.at[dst_all.at[b, j]], add=True)
            return carry

        lax.fori_loop(0, KB, body, 0)
    plsc.subcore_barrier()
    pltpu.sync_copy(deg_sh.at[pl.ds(r0, RPS)], out_hbm.at[cid, pl.ds(r0, RPS)])


@functools.partial(
    pl.kernel,
    mesh=_mesh,
    out_type=jax.ShapeDtypeStruct((NC, NP, D), jnp.float32),
    scratch_types=[
        pltpu.VMEM((KB, CH), jnp.int32),
        pltpu.VMEM((KB, CH), jnp.int32),
        pltpu.VMEM((KB, CH), jnp.int32),
        pltpu.VMEM((KB, CH), jnp.int32),
        pltpu.VMEM((CH, D), jnp.float32),
        pltpu.VMEM((CH, D), jnp.float32),
        pltpu.VMEM_SHARED((NP, D), jnp.float32),
        pltpu.SemaphoreType.DMA,
        pltpu.SemaphoreType.DMA,
        pltpu.SemaphoreType.DMA,
    ],
)
def _scatter_kernel(h_hbm, ei_hbm, out_hbm, src_a, dst_a, src_b, dst_b,
                    rows0, rows1, acc_sh, sem0, sem1, semi):
    cid = lax.axis_index("c")
    sid = lax.axis_index("s")
    wid = cid * NS + sid
    r0 = sid * RPS
    pltpu.sync_copy(h_hbm.at[pl.ds(r0, RPS)], acc_sh.at[pl.ds(r0, RPS)])
    plsc.subcore_barrier()

    pltpu.sync_copy(ei_hbm.at[0, wid, 0], src_a)
    pltpu.sync_copy(ei_hbm.at[1, wid, 0], dst_a)
    pltpu.async_copy(h_hbm.at[src_a.at[0]], rows0, sem0)
    pltpu.async_copy(h_hbm.at[src_a.at[1]], rows1, sem1)
    pltpu.async_copy(ei_hbm.at[0, wid, 1], src_b, semi)
    pltpu.async_copy(ei_hbm.at[1, wid, 1], dst_b, semi)

    for b in range(NB):
        sblk, dblk = (src_a, dst_a) if b % 2 == 0 else (src_b, dst_b)
        nsblk, ndblk = (src_b, dst_b) if b % 2 == 0 else (src_a, dst_a)

        def pair(i, carry, sblk=sblk, dblk=dblk):
            j0 = 2 * i
            pltpu.make_async_copy(h_hbm.at[sblk.at[j0]], rows0,
                                  sem0).wait()
            pltpu.sync_copy(rows0, acc_sh.at[dblk.at[j0]], add=True)
            pltpu.async_copy(h_hbm.at[sblk.at[j0 + 2]], rows0, sem0)
            j1 = j0 + 1
            pltpu.make_async_copy(h_hbm.at[sblk.at[j1]], rows1,
                                  sem1).wait()
            pltpu.sync_copy(rows1, acc_sh.at[dblk.at[j1]], add=True)
            pltpu.async_copy(h_hbm.at[sblk.at[j1 + 2]], rows1, sem1)
            return carry

        lax.fori_loop(0, KB // 2 - 1, pair, 0)
        jlast = KB - 2
        if b + 1 < NB:
            pltpu.make_async_copy(ei_hbm.at[0, wid, b + 1], nsblk,
                                  semi).wait()
            pltpu.make_async_copy(ei_hbm.at[1, wid, b + 1], ndblk,
                                  semi).wait()
        pltpu.make_async_copy(h_hbm.at[sblk.at[jlast]], rows0, sem0).wait()
        pltpu.sync_copy(rows0, acc_sh.at[dblk.at[jlast]], add=True)
        if b + 1 < NB:
            pltpu.async_copy(h_hbm.at[nsblk.at[0]], rows0, sem0)
        pltpu.make_async_copy(h_hbm.at[sblk.at[jlast + 1]], rows1,
                              sem1).wait()
        pltpu.sync_copy(rows1, acc_sh.at[dblk.at[jlast + 1]], add=True)
        if b + 1 < NB:
            pltpu.async_copy(h_hbm.at[nsblk.at[1]], rows1, sem1)
        if b + 2 < NB:
            pltpu.async_copy(ei_hbm.at[0, wid, b + 2], sblk, semi)
            pltpu.async_copy(ei_hbm.at[1, wid, b + 2], dblk, semi)
    plsc.subcore_barrier()
    pltpu.sync_copy(acc_sh.at[pl.ds(r0, RPS)], out_hbm.at[cid, pl.ds(r0, RPS)])


def _dinv_from_parts(p_ref):
    deg = p_ref[0, :, 0:1] + p_ref[1, :, 0:1] - 1.0
    return lax.rsqrt(deg)


def _matmul_pad_body(x_ref, w_ref, o_ref):
    h = jnp.dot(x_ref[...], w_ref[...], preferred_element_type=jnp.float32)
    o_ref[pl.ds(0, N), :] = h
    o_ref[pl.ds(N, NP - N), :] = jnp.zeros((NP - N, D), jnp.float32)


def _scale_body(h_ref, p_ref, o_ref):
    o_ref[...] = h_ref[...] * _dinv_from_parts(p_ref)


BR = 1280
GR = NP // BR

_row_spec = pl.BlockSpec((BR, D), lambda i: (i, 0))
_parts_spec = pl.BlockSpec((NC, BR, DEGW), lambda i: (0, i, 0))
_t_spec = pl.BlockSpec((NC, BR, D), lambda i: (0, i, 0))
_vec_spec = pl.BlockSpec((D,), lambda i: (0,))
_w_spec = pl.BlockSpec((D, D), lambda i: (0, 0))


def _layer2_body(t_ref, h_ref, p_ref, b_ref, w_ref, o_ref):
    dinv = _dinv_from_parts(p_ref)
    agg = t_ref[0] + t_ref[1] - h_ref[...]
    act = jnp.maximum(agg * dinv + b_ref[...], 0.0)
    o_ref[...] = jnp.dot(act, w_ref[...],
                         preferred_element_type=jnp.float32) * dinv


def _final_body(t_ref, h_ref, p_ref, b_ref, wfc_ref, bfc_ref, o_ref, s_ref):
    i = pl.program_id(0)
    dinv = _dinv_from_parts(p_ref)
    agg = t_ref[0] + t_ref[1] - h_ref[...]
    act = jnp.maximum(agg * dinv + b_ref[...], 0.0)
    row = lax.broadcasted_iota(jnp.int32, (BR, D), 0) + i * BR
    act = jnp.where(row < N, act, 0.0)
    part = jnp.sum(act, axis=0, keepdims=True)

    @pl.when(i == 0)
    def _():
        s_ref[...] = jnp.zeros((1, D), jnp.float32)

    s_ref[...] += part

    @pl.when(i == GR - 1)
    def _():
        m = s_ref[...] * (1.0 / N)
        o_ref[...] = jnp.dot(m, wfc_ref[...],
                             preferred_element_type=jnp.float32) + bfc_ref[...]


def kernel(x, edge_index, W1, b1, W2, b2, Wfc, bfc):
    ei = jnp.asarray(edge_index, jnp.int32).reshape(2, NW, NB, KB, CH)

    parts = _deg_kernel(ei)

    h1 = pl.pallas_call(
        _matmul_pad_body,
        out_shape=jax.ShapeDtypeStruct((NP, D), jnp.float32),
    )(x, W1)

    h1t = pl.pallas_call(
        _scale_body,
        grid=(GR,),
        in_specs=[_row_spec, _parts_spec],
        out_specs=_row_spec,
        out_shape=jax.ShapeDtypeStruct((NP, D), jnp.float32),
    )(h1, parts)

    t1 = _scatter_kernel(h1t, ei)

    h2t = pl.pallas_call(
        _layer2_body,
        grid=(GR,),
        in_specs=[_t_spec, _row_spec, _parts_spec, _vec_spec, _w_spec],
        out_specs=_row_spec,
        out_shape=jax.ShapeDtypeStruct((NP, D), jnp.float32),
    )(t1, h1t, parts, b1, W2)

    t2 = _scatter_kernel(h2t, ei)

    out = pl.pallas_call(
        _final_body,
        grid=(GR,),
        in_specs=[_t_spec, _row_spec, _parts_spec, _vec_spec, _w_spec,
                  _vec_spec],
        out_specs=pl.BlockSpec((1, D), lambda i: (0, 0)),
        out_shape=jax.ShapeDtypeStruct((1, D), jnp.float32),
        scratch_shapes=[pltpu.VMEM((1, D), jnp.float32)],
    )(t2, h2t, parts, b2, Wfc, bfc)
    return out
